# baseline (speedup 1.0000x reference)
import functools

import jax
import jax.numpy as jnp
from jax import lax
from jax.experimental import pallas as pl
from jax.experimental.pallas import tpu as pltpu
from jax.experimental.pallas import tpu_sc as plsc

_N = 10000
_E = 320000
_NC = 2
_NS = 16
_NW = _NC * _NS
_B = 128
_K = 80
_EPAD = _NW * _K * _B
_NPAD = 10112
_RPT = _NPAD // _NS

_mesh = plsc.VectorSubcoreMesh(
    core_axis_name="c", subcore_axis_name="s", num_cores=_NC, num_subcores=_NS
)


@functools.partial(
    pl.kernel,
    out_type=jax.ShapeDtypeStruct((_NC, _NPAD, 128), jnp.float32),
    mesh=_mesh,
    scratch_types=[
        pltpu.VMEM((_K, _B), jnp.int32),
        pltpu.VMEM((_B, 128), jnp.float32),
        pltpu.VMEM_SHARED((_NPAD, 128), jnp.float32),
    ],
)
def _deg_sc(dst_hbm, ones_hbm, zeros_hbm, out_hbm, dst_v, ones_v, deg_sh):
    c = lax.axis_index("c")
    s = lax.axis_index("s")
    wid = c * _NS + s
    pltpu.sync_copy(zeros_hbm, deg_sh.at[pl.ds(s * _RPT, _RPT)])
    pltpu.sync_copy(dst_hbm.at[wid], dst_v)
    pltpu.sync_copy(ones_hbm, ones_v)
    plsc.subcore_barrier()

    def step(j, carry):
        pltpu.sync_copy(ones_v, deg_sh.at[dst_v.at[j]], add=True)
        return carry

    lax.fori_loop(0, _K, step, 0)
    plsc.subcore_barrier()
    pltpu.sync_copy(
        deg_sh.at[pl.ds(s * _RPT, _RPT)], out_hbm.at[c, pl.ds(s * _RPT, _RPT)]
    )


def _make_edge_agg(width):
    @functools.partial(
        pl.kernel,
        out_type=jax.ShapeDtypeStruct((_NC, _NPAD, width), jnp.float32),
        mesh=_mesh,
        scratch_types=[
            pltpu.VMEM((_K, _B), jnp.int32),
            pltpu.VMEM((16, _B), jnp.int32),
            pltpu.VMEM((_B, width), jnp.float32),
            pltpu.VMEM((_B, width), jnp.float32),
            pltpu.VMEM_SHARED((_NPAD, width), jnp.float32),
            pltpu.SemaphoreType.DMA,
            pltpu.SemaphoreType.DMA,
            pltpu.SemaphoreType.DMA,
            pltpu.SemaphoreType.DMA,
            pltpu.SemaphoreType.DMA,
            pltpu.SemaphoreType.DMA,
        ],
    )
    def edge_agg(y_hbm, src_hbm, dst_hbm, zeros_hbm, out_hbm,
                 src_v, dch_v, rows0_v, rows1_v, agg_sh,
                 gsem0, gsem1, ssem0, ssem1, isem0, isem1):
        nch = _K // 8
        c = lax.axis_index("c")
        s = lax.axis_index("s")
        wid = c * _NS + s
        pltpu.sync_copy(zeros_hbm, agg_sh.at[pl.ds(s * _RPT, _RPT)])
        pltpu.sync_copy(src_hbm.at[wid], src_v)
        plsc.subcore_barrier()

        bufs = (rows0_v, rows1_v)
        gsems = (gsem0, gsem1)
        ssems = (ssem0, ssem1)
        isems = (isem0, isem1)

        def dchunk(m):
            return dst_hbm.at[wid, pl.ds(pl.multiple_of(8 * m, 8), 8)]

        def gather(j, p):
            pltpu.async_copy(y_hbm.at[src_v.at[j]], bufs[p], gsems[p])

        def gwait(j, p):
            pltpu.make_async_copy(y_hbm.at[src_v.at[j]], bufs[p], gsems[p]).wait()

        def scat(row, p):
            pltpu.async_copy(bufs[p], agg_sh.at[dch_v.at[row]], ssems[p],
                             add=True)

        def swait(p):
            pltpu.make_async_copy(
                bufs[p], agg_sh.at[dch_v.at[0]], ssems[p]).wait()

        for q in range(2):
            pltpu.async_copy(dchunk(q), dch_v.at[pl.ds(8 * q, 8)], isems[q])
        gather(0, 0)

        for q in range(2):
            pltpu.make_async_copy(
                dchunk(q), dch_v.at[pl.ds(8 * q, 8)], isems[q]).wait()
            for t in range(8):
                j = 8 * q + t
                p = t % 2
                gwait(j, p)
                scat(8 * q + t, p)
                if j >= 1:
                    swait(1 - p)
                gather(j + 1, 1 - p)
            pltpu.async_copy(dchunk(q + 2), dch_v.at[pl.ds(8 * q, 8)], isems[q])

        def step(i, carry):
            for q in range(2):
                m = 2 * i + q
                pltpu.make_async_copy(
                    dchunk(m), dch_v.at[pl.ds(8 * q, 8)], isems[q]).wait()
                for t in range(8):
                    j = 8 * m + t
                    p = t % 2
                    gwait(j, p)
                    scat(8 * q + t, p)
                    swait(1 - p)
                    jn = jnp.minimum(j + 1, _K - 1)
                    gather(jn, 1 - p)
                mn = jnp.minimum(m + 2, nch - 1)
                pltpu.async_copy(dchunk(mn), dch_v.at[pl.ds(8 * q, 8)], isems[q])
            return carry

        lax.fori_loop(1, nch // 2, step, 0)
        gwait(_K - 1, 0)
        swait(1)
        for q in range(2):
            pltpu.make_async_copy(
                dchunk(nch - 1), dch_v.at[pl.ds(8 * q, 8)], isems[q]).wait()
        plsc.subcore_barrier()
        pltpu.sync_copy(
            agg_sh.at[pl.ds(s * _RPT, _RPT)], out_hbm.at[c, pl.ds(s * _RPT, _RPT)]
        )

    return edge_agg


_edge_agg_128 = _make_edge_agg(128)


_BN = 1000


def _dis_col(d_ref):
    deg = d_ref[0, :, 0:1] + d_ref[1, :, 0:1] + 1.0
    return lax.rsqrt(deg)


def _first_body(x_ref, w_ref, d_ref, o_ref):
    dis = _dis_col(d_ref)
    o_ref[...] = dis * jnp.dot(
        x_ref[...], w_ref[...], preferred_element_type=jnp.float32
    )


def _mid_body(a_ref, y_ref, d_ref, b_ref, w_ref, o_ref):
    dis = _dis_col(d_ref)
    h = jnp.maximum(
        dis * (a_ref[0] + a_ref[1] + y_ref[...]) + b_ref[...][0:1, :], 0.0
    )
    o_ref[...] = dis * jnp.dot(h, w_ref[...], preferred_element_type=jnp.float32)


def _premul_body(a_ref, y_ref, d_ref, b_ref, o_ref):
    dis = _dis_col(d_ref)
    h = jnp.maximum(
        dis * (a_ref[0] + a_ref[1] + y_ref[...]) + b_ref[...][0:1, :], 0.0
    )
    o_ref[...] = dis * h


def _final_body(a_ref, z_ref, d_ref, b_ref, w_ref, o_ref):
    dis = _dis_col(d_ref)
    zsum = a_ref[0] + a_ref[1] + z_ref[...]
    o_ref[...] = (
        dis * jnp.dot(zsum, w_ref[...], preferred_element_type=jnp.float32)
        + b_ref[...][0:1, :]
    )


def _row_spec(w):
    return pl.BlockSpec((_BN, w), lambda i: (i, 0))


def _pair_spec(w):
    return pl.BlockSpec((2, _BN, w), lambda i: (0, i, 0))


def _full_spec(r, ccols):
    return pl.BlockSpec((r, ccols), lambda i: (0, 0))


def _tc_first(x, w, d):
    return pl.pallas_call(
        _first_body,
        grid=(_N // _BN,),
        in_specs=[_row_spec(128), _full_spec(128, 128), _pair_spec(128)],
        out_specs=_row_spec(128),
        out_shape=jax.ShapeDtypeStruct((_N, 128), jnp.float32),
    )(x, w, d)


def _tc_mid(a, y, d, b8, w, wout):
    return pl.pallas_call(
        _mid_body,
        grid=(_N // _BN,),
        in_specs=[
            _pair_spec(128), _row_spec(128), _pair_spec(128),
            _full_spec(8, 128), _full_spec(128, wout),
        ],
        out_specs=_row_spec(wout),
        out_shape=jax.ShapeDtypeStruct((_N, wout), jnp.float32),
    )(a, y, d, b8, w)


def _tc_premul(a, y, d, b8):
    return pl.pallas_call(
        _premul_body,
        grid=(_N // _BN,),
        in_specs=[
            _pair_spec(128), _row_spec(128), _pair_spec(128),
            _full_spec(8, 128),
        ],
        out_specs=_row_spec(128),
        out_shape=jax.ShapeDtypeStruct((_N, 128), jnp.float32),
    )(a, y, d, b8)


def _tc_final(a, z, d, b8, w):
    return pl.pallas_call(
        _final_body,
        grid=(_N // _BN,),
        in_specs=[
            _pair_spec(128), _row_spec(128), _pair_spec(128),
            _full_spec(8, 64), _full_spec(128, 64),
        ],
        out_specs=_row_spec(64),
        out_shape=jax.ShapeDtypeStruct((_N, 64), jnp.float32),
    )(a, z, d, b8, w)


def kernel(x, edge_index, W1, b1, W2, b2, W3, b3):
    src = edge_index[0]
    dst = edge_index[1]
    pad = _EPAD - _E
    pi = jnp.arange(pad, dtype=jnp.int32)
    src_p = jnp.concatenate([src, pi % _N]).reshape(_NW, _K, _B)
    dst_p = jnp.concatenate(
        [dst, _N + pi % (_NPAD - _N)]).reshape(_NW, _K, _B)

    ones128 = jnp.ones((_B, 128), jnp.float32)
    z128 = jnp.zeros((_RPT, 128), jnp.float32)

    degp = _deg_sc(dst_p, ones128, z128)

    b1w = jnp.broadcast_to(b1.reshape(1, -1), (8, 128))
    b2w = jnp.broadcast_to(b2.reshape(1, -1), (8, 128))
    b3w = jnp.broadcast_to(b3.reshape(1, -1), (8, 64))

    y1 = _tc_first(x, W1, degp)
    a1 = _edge_agg_128(y1, src_p, dst_p, z128)
    y2 = _tc_mid(a1, y1, degp, b1w, W2, 128)
    a2 = _edge_agg_128(y2, src_p, dst_p, z128)
    z = _tc_premul(a2, y2, degp, b2w)
    a3 = _edge_agg_128(z, src_p, dst_p, z128)
    return _tc_final(a3, z, degp, b3w, W3)


# speedup vs baseline: 1.1314x; 1.1314x over previous
import functools

import jax
import jax.numpy as jnp
from jax import lax
from jax.experimental import pallas as pl
from jax.experimental.pallas import tpu as pltpu
from jax.experimental.pallas import tpu_sc as plsc

_N = 10000
_E = 320000
_NC = 2
_NS = 16
_NW = _NC * _NS
_B = 128
_K = 80
_EPAD = _NW * _K * _B
_NPAD = 10112
_RPT = _NPAD // _NS

_mesh = plsc.VectorSubcoreMesh(
    core_axis_name="c", subcore_axis_name="s", num_cores=_NC, num_subcores=_NS
)


@functools.partial(
    pl.kernel,
    out_type=jax.ShapeDtypeStruct((_NC, _NPAD, 128), jnp.float32),
    mesh=_mesh,
    scratch_types=[
        pltpu.VMEM((_K, _B), jnp.int32),
        pltpu.VMEM((_B, 128), jnp.float32),
        pltpu.VMEM_SHARED((_NPAD, 128), jnp.float32),
    ],
)
def _deg_sc(dst_hbm, ones_hbm, zeros_hbm, out_hbm, dst_v, ones_v, deg_sh):
    c = lax.axis_index("c")
    s = lax.axis_index("s")
    wid = c * _NS + s
    pltpu.sync_copy(zeros_hbm, deg_sh.at[pl.ds(s * _RPT, _RPT)])
    pltpu.sync_copy(dst_hbm.at[wid], dst_v)
    pltpu.sync_copy(ones_hbm, ones_v)
    plsc.subcore_barrier()

    def step(j, carry):
        pltpu.sync_copy(ones_v, deg_sh.at[dst_v.at[j]], add=True)
        return carry

    lax.fori_loop(0, _K, step, 0)
    plsc.subcore_barrier()
    pltpu.sync_copy(
        deg_sh.at[pl.ds(s * _RPT, _RPT)], out_hbm.at[c, pl.ds(s * _RPT, _RPT)]
    )


def _make_edge_agg(width):
    @functools.partial(
        pl.kernel,
        out_type=jax.ShapeDtypeStruct((_NC, _NPAD, width), jnp.float32),
        mesh=_mesh,
        scratch_types=[
            pltpu.VMEM((_K, _B), jnp.int32),
            pltpu.VMEM((16, _B), jnp.int32),
            pltpu.VMEM((_B, width), jnp.float32),
            pltpu.VMEM((_B, width), jnp.float32),
            pltpu.VMEM_SHARED((_NPAD, width), jnp.float32),
            pltpu.SemaphoreType.DMA,
            pltpu.SemaphoreType.DMA,
            pltpu.SemaphoreType.DMA,
            pltpu.SemaphoreType.DMA,
        ],
    )
    def edge_agg(y_hbm, src_hbm, dst_hbm, zeros_hbm, out_hbm,
                 src_v, dch_v, rows0_v, rows1_v, agg_sh,
                 gsem0, gsem1, isem0, isem1):
        nch = _K // 8
        c = lax.axis_index("c")
        s = lax.axis_index("s")
        wid = c * _NS + s
        pltpu.sync_copy(zeros_hbm, agg_sh.at[pl.ds(s * _RPT, _RPT)])
        pltpu.sync_copy(src_hbm.at[wid], src_v)
        plsc.subcore_barrier()

        bufs = (rows0_v, rows1_v)
        gsems = (gsem0, gsem1)
        isems = (isem0, isem1)

        def dchunk(m):
            return dst_hbm.at[wid, pl.ds(pl.multiple_of(8 * m, 8), 8)]

        for q in range(2):
            pltpu.async_copy(dchunk(q), dch_v.at[pl.ds(8 * q, 8)], isems[q])
            pltpu.async_copy(y_hbm.at[src_v.at[q]], bufs[q], gsems[q])

        def step(i, carry):
            for q in range(2):
                m = 2 * i + q
                pltpu.make_async_copy(
                    dchunk(m), dch_v.at[pl.ds(8 * q, 8)], isems[q]).wait()
                for t in range(8):
                    j = 8 * m + t
                    p = t % 2
                    pltpu.make_async_copy(
                        y_hbm.at[src_v.at[j]], bufs[p], gsems[p]).wait()
                    pltpu.sync_copy(
                        bufs[p], agg_sh.at[dch_v.at[8 * q + t]], add=True)
                    jn = jnp.minimum(j + 2, _K - 1)
                    pltpu.async_copy(y_hbm.at[src_v.at[jn]], bufs[p], gsems[p])
                mn = jnp.minimum(m + 2, nch - 1)
                pltpu.async_copy(dchunk(mn), dch_v.at[pl.ds(8 * q, 8)], isems[q])
            return carry

        lax.fori_loop(0, nch // 2, step, 0)
        for q in range(2):
            pltpu.make_async_copy(
                y_hbm.at[src_v.at[_K - 1]], bufs[q], gsems[q]).wait()
            pltpu.make_async_copy(
                dchunk(nch - 1), dch_v.at[pl.ds(8 * q, 8)], isems[q]).wait()
        plsc.subcore_barrier()
        pltpu.sync_copy(
            agg_sh.at[pl.ds(s * _RPT, _RPT)], out_hbm.at[c, pl.ds(s * _RPT, _RPT)]
        )

    return edge_agg


_edge_agg_128 = _make_edge_agg(128)


_BN = 1000


def _dis_col(d_ref):
    deg = d_ref[0, :, 0:1] + d_ref[1, :, 0:1] + 1.0
    return lax.rsqrt(deg)


def _first_body(x_ref, w_ref, d_ref, o_ref):
    dis = _dis_col(d_ref)
    o_ref[...] = dis * jnp.dot(
        x_ref[...], w_ref[...], preferred_element_type=jnp.float32
    )


def _mid_body(a_ref, y_ref, d_ref, b_ref, w_ref, o_ref):
    dis = _dis_col(d_ref)
    h = jnp.maximum(
        dis * (a_ref[0] + a_ref[1] + y_ref[...]) + b_ref[...][0:1, :], 0.0
    )
    o_ref[...] = dis * jnp.dot(h, w_ref[...], preferred_element_type=jnp.float32)


def _premul_body(a_ref, y_ref, d_ref, b_ref, o_ref):
    dis = _dis_col(d_ref)
    h = jnp.maximum(
        dis * (a_ref[0] + a_ref[1] + y_ref[...]) + b_ref[...][0:1, :], 0.0
    )
    o_ref[...] = dis * h


def _final_body(a_ref, z_ref, d_ref, b_ref, w_ref, o_ref):
    dis = _dis_col(d_ref)
    zsum = a_ref[0] + a_ref[1] + z_ref[...]
    o_ref[...] = (
        dis * jnp.dot(zsum, w_ref[...], preferred_element_type=jnp.float32)
        + b_ref[...][0:1, :]
    )


def _row_spec(w):
    return pl.BlockSpec((_BN, w), lambda i: (i, 0))


def _pair_spec(w):
    return pl.BlockSpec((2, _BN, w), lambda i: (0, i, 0))


def _full_spec(r, ccols):
    return pl.BlockSpec((r, ccols), lambda i: (0, 0))


def _tc_first(x, w, d):
    return pl.pallas_call(
        _first_body,
        grid=(_N // _BN,),
        in_specs=[_row_spec(128), _full_spec(128, 128), _pair_spec(128)],
        out_specs=_row_spec(128),
        out_shape=jax.ShapeDtypeStruct((_N, 128), jnp.float32),
    )(x, w, d)


def _tc_mid(a, y, d, b8, w, wout):
    return pl.pallas_call(
        _mid_body,
        grid=(_N // _BN,),
        in_specs=[
            _pair_spec(128), _row_spec(128), _pair_spec(128),
            _full_spec(8, 128), _full_spec(128, wout),
        ],
        out_specs=_row_spec(wout),
        out_shape=jax.ShapeDtypeStruct((_N, wout), jnp.float32),
    )(a, y, d, b8, w)


def _tc_premul(a, y, d, b8):
    return pl.pallas_call(
        _premul_body,
        grid=(_N // _BN,),
        in_specs=[
            _pair_spec(128), _row_spec(128), _pair_spec(128),
            _full_spec(8, 128),
        ],
        out_specs=_row_spec(128),
        out_shape=jax.ShapeDtypeStruct((_N, 128), jnp.float32),
    )(a, y, d, b8)


def _tc_final(a, z, d, b8, w):
    return pl.pallas_call(
        _final_body,
        grid=(_N // _BN,),
        in_specs=[
            _pair_spec(128), _row_spec(128), _pair_spec(128),
            _full_spec(8, 64), _full_spec(128, 64),
        ],
        out_specs=_row_spec(64),
        out_shape=jax.ShapeDtypeStruct((_N, 64), jnp.float32),
    )(a, z, d, b8, w)


def kernel(x, edge_index, W1, b1, W2, b2, W3, b3):
    src = edge_index[0]
    dst = edge_index[1]
    pad = _EPAD - _E
    pi = jnp.arange(pad, dtype=jnp.int32)
    src_p = jnp.concatenate([src, pi % _N]).reshape(_NW, _K, _B)
    dst_p = jnp.concatenate(
        [dst, _N + pi % (_NPAD - _N)]).reshape(_NW, _K, _B)

    ones128 = jnp.ones((_B, 128), jnp.float32)
    z128 = jnp.zeros((_RPT, 128), jnp.float32)

    degp = _deg_sc(dst_p, ones128, z128)

    b1w = jnp.broadcast_to(b1.reshape(1, -1), (8, 128))
    b2w = jnp.broadcast_to(b2.reshape(1, -1), (8, 128))
    b3w = jnp.broadcast_to(b3.reshape(1, -1), (8, 64))

    y1 = _tc_first(x, W1, degp)
    a1 = _edge_agg_128(y1, src_p, dst_p, z128)
    y2 = _tc_mid(a1, y1, degp, b1w, W2, 128)
    a2 = _edge_agg_128(y2, src_p, dst_p, z128)
    z = _tc_premul(a2, y2, degp, b2w)
    a3 = _edge_agg_128(z, src_p, dst_p, z128)
    return _tc_final(a3, z, degp, b3w, W3)
